# Initial kernel scaffold; baseline (speedup 1.0000x reference)
import jax
import jax.numpy as jnp
from jax.experimental import pallas as pl


def kernel(x, edge_index_ac, edge_index_tf, branch_vals_ac, branch_vals_tf, W_enc, b_enc, W_e_ac, W_e_tf, msg_W, msg_b, node_W, node_b, W_dec, b_dec):
    raise NotImplementedError("write your pallas kernel here")


# speedup vs baseline: 2.0002x; 2.0002x over previous
import functools

import jax
import jax.numpy as jnp
from jax import lax
from jax.experimental import pallas as pl
from jax.experimental.pallas import tpu as pltpu
from jax.experimental.pallas import tpu_sc as plsc

H = 128
NTILES = 16
NCORES = 2


def _prep_body(x_ref, we_ref, be_ref, nodes_ref):
    nodes_ref[...] = jnp.maximum(
        jnp.dot(x_ref[...], we_ref[...],
                preferred_element_type=jnp.float32) + be_ref[...], 0.0)


def _tc_prep(x, w_enc, b_enc, blk):
    n = x.shape[0]
    return pl.pallas_call(
        _prep_body,
        grid=(n // blk,),
        in_specs=[
            pl.BlockSpec((blk, H), lambda i: (i, 0)),
            pl.BlockSpec((H, H), lambda i: (0, 0)),
            pl.BlockSpec((1, H), lambda i: (0, 0)),
        ],
        out_specs=pl.BlockSpec((blk, H), lambda i: (i, 0)),
        out_shape=jax.ShapeDtypeStruct((n, H), jnp.float32),
    )(x, w_enc, b_enc.reshape(1, H))


def _enc_body(bv_ref, w_ref, out_ref):
    out_ref[...] = jnp.maximum(
        jnp.dot(bv_ref[...], w_ref[...],
                preferred_element_type=jnp.float32), 0.0)


def _tc_enc(bv, w_e, blk):
    e, f = bv.shape
    return pl.pallas_call(
        _enc_body,
        grid=(e // blk,),
        in_specs=[
            pl.BlockSpec((blk, f), lambda i: (i, 0)),
            pl.BlockSpec((f, H), lambda i: (0, 0)),
        ],
        out_specs=pl.BlockSpec((blk, H), lambda i: (i, 0)),
        out_shape=jax.ShapeDtypeStruct((e, H), jnp.float32),
    )(bv, w_e)


def _edge_body(e_ref, g_ref, w12_ref, w3_ref, b_ref, out_ref):
    e0 = e_ref[...]
    m = jnp.dot(g_ref[...], w12_ref[...], preferred_element_type=jnp.float32) \
        + jnp.dot(e0, w3_ref[...], preferred_element_type=jnp.float32)
    out_ref[...] = e0 + jnp.maximum(m + b_ref[...], 0.0)


def _tc_edge(edges, g256, w12, w3, b, blk):
    e = edges.shape[0]
    return pl.pallas_call(
        _edge_body,
        grid=(e // blk,),
        in_specs=[
            pl.BlockSpec((blk, H), lambda i: (i, 0)),
            pl.BlockSpec((blk, 2 * H), lambda i: (i, 0)),
            pl.BlockSpec((2 * H, H), lambda i: (0, 0)),
            pl.BlockSpec((H, H), lambda i: (0, 0)),
            pl.BlockSpec((1, H), lambda i: (0, 0)),
        ],
        out_specs=pl.BlockSpec((blk, H), lambda i: (i, 0)),
        out_shape=jax.ShapeDtypeStruct((e, H), jnp.float32),
    )(edges, g256, w12, w3, b.reshape(1, H))


def _node_body(n_ref, ni_ref, wn_ref, b_ref, nout_ref):
    nout_ref[...] = n_ref[...] + jnp.maximum(
        jnp.dot(ni_ref[...], wn_ref[...],
                preferred_element_type=jnp.float32) + b_ref[...], 0.0)


def _tc_node(nodes, node_in, wn, b, blk):
    n = nodes.shape[0]
    return pl.pallas_call(
        _node_body,
        grid=(n // blk,),
        in_specs=[
            pl.BlockSpec((blk, H), lambda i: (i, 0)),
            pl.BlockSpec((blk, 2 * H), lambda i: (i, 0)),
            pl.BlockSpec((2 * H, H), lambda i: (0, 0)),
            pl.BlockSpec((1, H), lambda i: (0, 0)),
        ],
        out_specs=pl.BlockSpec((blk, H), lambda i: (i, 0)),
        out_shape=jax.ShapeDtypeStruct((n, H), jnp.float32),
    )(nodes, node_in, wn, b.reshape(1, H))


def _node_last_body(n_ref, ni_ref, wn_ref, b_ref, wdec_ref, bdec_ref,
                    bus_ref, vtab_ref):
    n2 = n_ref[...] + jnp.maximum(
        jnp.dot(ni_ref[...], wn_ref[...],
                preferred_element_type=jnp.float32) + b_ref[...], 0.0)
    bus = jnp.dot(n2, wdec_ref[...], preferred_element_type=jnp.float32) \
        + bdec_ref[...]
    bus_ref[...] = bus
    va = bus[:, 0:1]
    vm = bus[:, 3:4]
    vre = vm * jnp.cos(va)
    vim = vm * jnp.sin(va)
    blk = vre.shape[0]
    vtab_ref[...] = jnp.concatenate(
        [vre, vim, jnp.zeros((blk, 126), jnp.float32)], axis=1)


def _tc_node_last(nodes, node_in, wn, b, w_dec, b_dec, blk):
    n = nodes.shape[0]
    return pl.pallas_call(
        _node_last_body,
        grid=(n // blk,),
        in_specs=[
            pl.BlockSpec((blk, H), lambda i: (i, 0)),
            pl.BlockSpec((blk, 2 * H), lambda i: (i, 0)),
            pl.BlockSpec((2 * H, H), lambda i: (0, 0)),
            pl.BlockSpec((1, H), lambda i: (0, 0)),
            pl.BlockSpec((H, 4), lambda i: (0, 0)),
            pl.BlockSpec((1, 4), lambda i: (0, 0)),
        ],
        out_specs=[
            pl.BlockSpec((blk, 4), lambda i: (i, 0)),
            pl.BlockSpec((blk, H), lambda i: (i, 0)),
        ],
        out_shape=[
            jax.ShapeDtypeStruct((n, 4), jnp.float32),
            jax.ShapeDtypeStruct((n, H), jnp.float32),
        ],
    )(nodes, node_in, wn, b.reshape(1, H), w_dec, b_dec.reshape(1, 4))


def _flow_body(v_ref, par_ref, out_ref):
    vi = v_ref[0]
    vj = v_ref[1]
    vi_re, vi_im = vi[:, 0:1], vi[:, 1:2]
    vj_re, vj_im = vj[:, 0:1], vj[:, 1:2]
    r = par_ref[:, 0:1]
    xx = par_ref[:, 1:2]
    b_fr = par_ref[:, 2:3]
    b_to = par_ref[:, 3:4]
    tap = par_ref[:, 4:5]
    shift = par_ref[:, 5:6]
    den = r * r + xx * xx
    y_re = r / den
    y_im = -xx / den
    cs = jnp.cos(shift)
    sn = jnp.sin(shift)
    t2 = tap * tap
    ai2 = vi_re * vi_re + vi_im * vi_im
    aj2 = vj_re * vj_re + vj_im * vj_im
    m_re = vi_re * vj_re + vi_im * vj_im
    m_im = vi_im * vj_re - vi_re * vj_im
    cy_m_re = y_re * m_re + y_im * m_im
    cy_m_im = y_re * m_im - y_im * m_re
    term_re = (cy_m_re * cs + cy_m_im * sn) / tap
    term_im = (cy_m_im * cs - cy_m_re * sn) / tap
    p_fr = y_re * ai2 / t2 - term_re
    q_fr = -(y_im + b_fr) * ai2 / t2 - term_im
    cy_n_re = y_re * m_re - y_im * m_im
    cy_n_im = -y_re * m_im - y_im * m_re
    term2_re = (cy_n_re * cs - cy_n_im * sn) / tap
    term2_im = (cy_n_im * cs + cy_n_re * sn) / tap
    p_to = y_re * aj2 - term2_re
    q_to = -(y_im + b_to) * aj2 - term2_im
    out_ref[...] = jnp.concatenate([p_to, q_to, p_fr, q_fr], axis=1)


def _tc_flow(v3, params, blk):
    e = params.shape[0]
    return pl.pallas_call(
        _flow_body,
        grid=(e // blk,),
        in_specs=[
            pl.BlockSpec((2, blk, H), lambda i: (0, i, 0)),
            pl.BlockSpec((blk, 8), lambda i: (i, 0)),
        ],
        out_specs=pl.BlockSpec((blk, 4), lambda i: (i, 0)),
        out_shape=jax.ShapeDtypeStruct((e, 4), jnp.float32),
    )(v3, params)


def _make_sc_gather(n_rows, d, e_total, cols_mode):
    ept = e_total // NTILES
    nb = ept // 128
    tail = ept - nb * 128
    stride = (n_rows // NTILES) // 8 * 8
    size = n_rows - stride * (NTILES - 1)
    assert size >= n_rows // NTILES and size % 8 == 0
    mesh = plsc.VectorSubcoreMesh(core_axis_name="c", subcore_axis_name="s")

    if cols_mode:
        out_t = jax.ShapeDtypeStruct((e_total, 2 * d), jnp.float32)
    else:
        out_t = jax.ShapeDtypeStruct((2 * e_total, d), jnp.float32)

    scratch = [
        pltpu.VMEM_SHARED((n_rows, d), jnp.float32),
        pltpu.VMEM((128,), jnp.int32),
        pltpu.VMEM((128, d), jnp.float32),
        pltpu.SemaphoreType.DMA,
    ]
    if tail:
        scratch += [
            pltpu.VMEM((tail,), jnp.int32),
            pltpu.VMEM((tail, d), jnp.float32),
        ]

    @functools.partial(
        pl.kernel,
        out_type=out_t,
        mesh=mesh,
        scratch_types=scratch,
    )
    def k(tab_hbm, idx_hbm, out_hbm, shared, idx_v, rows_v, sem, *tl):
        c = lax.axis_index("c")
        s = lax.axis_index("s")
        pltpu.sync_copy(
            tab_hbm.at[pl.ds(s * stride, size), :],
            shared.at[pl.ds(s * stride, size), :])
        plsc.subcore_barrier()
        base = c * e_total + s * ept

        def store(rows, b0, cnt):
            erow = b0 - c * e_total
            if cols_mode:
                pltpu.sync_copy(
                    rows, out_hbm.at[pl.ds(erow, cnt), pl.ds(c * d, d)])
            else:
                pltpu.sync_copy(rows, out_hbm.at[pl.ds(b0, cnt), :])

        def body(it, _):
            b0 = base + it * 128
            pltpu.sync_copy(idx_hbm.at[pl.ds(b0, 128)], idx_v)
            pltpu.async_copy(shared.at[idx_v], rows_v, sem).wait()
            store(rows_v, b0, 128)
            return 0

        lax.fori_loop(0, nb, body, 0, unroll=False)
        if tail:
            idx_t, rows_t = tl
            b0 = base + nb * 128
            pltpu.sync_copy(idx_hbm.at[pl.ds(b0, tail)], idx_t)
            pltpu.async_copy(shared.at[idx_t], rows_t, sem).wait()
            store(rows_t, b0, tail)

    return k


def kernel(x, edge_index_ac, edge_index_tf, branch_vals_ac, branch_vals_tf,
           W_enc, b_enc, W_e_ac, W_e_tf, msg_W, msg_b, node_W, node_b,
           W_dec, b_dec):
    n = x.shape[0]
    e_ac = branch_vals_ac.shape[0]
    e_tf = branch_vals_tf.shape[0]
    e = e_ac + e_tf
    kk = msg_W.shape[0]

    ei = jnp.concatenate([edge_index_ac, edge_index_tf], axis=1)
    idx_flat = ei.reshape(2 * e)
    dst = ei[1]
    params = jnp.stack([
        jnp.concatenate([branch_vals_ac[:, 4], branch_vals_tf[:, 2]]),
        jnp.concatenate([branch_vals_ac[:, 5], branch_vals_tf[:, 3]]),
        jnp.concatenate([branch_vals_ac[:, 2], branch_vals_tf[:, 9]]),
        jnp.concatenate([branch_vals_ac[:, 3], branch_vals_tf[:, 10]]),
        jnp.concatenate([jnp.ones((e_ac,), jnp.float32),
                         branch_vals_tf[:, 7]]),
        jnp.concatenate([jnp.zeros((e_ac,), jnp.float32),
                         branch_vals_tf[:, 8]]),
        jnp.zeros((e,), jnp.float32),
        jnp.zeros((e,), jnp.float32),
    ], axis=1)

    gather_nodes = _make_sc_gather(n, H, e, cols_mode=True)
    gather_v = _make_sc_gather(n, H, e, cols_mode=False)

    nodes = _tc_prep(x, W_enc, b_enc, 1000)
    edges = jnp.concatenate([
        _tc_enc(branch_vals_ac, W_e_ac, 512),
        _tc_enc(branch_vals_tf, W_e_tf, 512),
    ], axis=0)
    bus_out = vtab = None
    for l in range(kk):
        g256 = gather_nodes(nodes, idx_flat)
        edges = _tc_edge(edges, g256, msg_W[l, :2 * H, :],
                         msg_W[l, 2 * H:, :], msg_b[l], 512)
        agg = jax.ops.segment_sum(edges, dst, num_segments=n)
        node_in = jnp.concatenate([nodes, agg], axis=1)
        if l < kk - 1:
            nodes = _tc_node(nodes, node_in, node_W[l], node_b[l], 1000)
        else:
            bus_out, vtab = _tc_node_last(nodes, node_in, node_W[l],
                                          node_b[l], W_dec, b_dec, 1000)
    vij = gather_v(vtab, idx_flat)
    edge_preds = _tc_flow(vij.reshape(2, e, H), params, 2000)
    return bus_out, edge_preds


# speedup vs baseline: 2.1410x; 1.0704x over previous
import functools

import jax
import jax.numpy as jnp
from jax import lax
from jax.experimental import pallas as pl
from jax.experimental.pallas import tpu as pltpu
from jax.experimental.pallas import tpu_sc as plsc

H = 128
NTILES = 16
NCORES = 2


def _prep_body(x_ref, we_ref, be_ref, nodes_ref):
    nodes_ref[...] = jnp.maximum(
        jnp.dot(x_ref[...], we_ref[...],
                preferred_element_type=jnp.float32) + be_ref[...], 0.0)


def _tc_prep(x, w_enc, b_enc, blk):
    n = x.shape[0]
    return pl.pallas_call(
        _prep_body,
        grid=(n // blk,),
        in_specs=[
            pl.BlockSpec((blk, H), lambda i: (i, 0)),
            pl.BlockSpec((H, H), lambda i: (0, 0)),
            pl.BlockSpec((1, H), lambda i: (0, 0)),
        ],
        out_specs=pl.BlockSpec((blk, H), lambda i: (i, 0)),
        out_shape=jax.ShapeDtypeStruct((n, H), jnp.float32),
    )(x, w_enc, b_enc.reshape(1, H))


def _enc_body(bv_ref, w_ref, out_ref):
    out_ref[...] = jnp.maximum(
        jnp.dot(bv_ref[...], w_ref[...],
                preferred_element_type=jnp.float32), 0.0)


def _tc_enc(bv, w_e, blk):
    e, f = bv.shape
    return pl.pallas_call(
        _enc_body,
        grid=(e // blk,),
        in_specs=[
            pl.BlockSpec((blk, f), lambda i: (i, 0)),
            pl.BlockSpec((f, H), lambda i: (0, 0)),
        ],
        out_specs=pl.BlockSpec((blk, H), lambda i: (i, 0)),
        out_shape=jax.ShapeDtypeStruct((e, H), jnp.float32),
    )(bv, w_e)


def _edge_body(e_ref, g_ref, w12_ref, w3_ref, b_ref, out_ref):
    e0 = e_ref[...]
    m = jnp.dot(g_ref[...], w12_ref[...], preferred_element_type=jnp.float32) \
        + jnp.dot(e0, w3_ref[...], preferred_element_type=jnp.float32)
    out_ref[...] = e0 + jnp.maximum(m + b_ref[...], 0.0)


def _tc_edge(edges, g256, w12, w3, b, blk):
    e = edges.shape[0]
    return pl.pallas_call(
        _edge_body,
        grid=(e // blk,),
        in_specs=[
            pl.BlockSpec((blk, H), lambda i: (i, 0)),
            pl.BlockSpec((blk, 2 * H), lambda i: (i, 0)),
            pl.BlockSpec((2 * H, H), lambda i: (0, 0)),
            pl.BlockSpec((H, H), lambda i: (0, 0)),
            pl.BlockSpec((1, H), lambda i: (0, 0)),
        ],
        out_specs=pl.BlockSpec((blk, H), lambda i: (i, 0)),
        out_shape=jax.ShapeDtypeStruct((e, H), jnp.float32),
    )(edges, g256, w12, w3, b.reshape(1, H))


def _node_body(n_ref, ni_ref, wn_ref, b_ref, nout_ref):
    nout_ref[...] = n_ref[...] + jnp.maximum(
        jnp.dot(ni_ref[...], wn_ref[...],
                preferred_element_type=jnp.float32) + b_ref[...], 0.0)


def _tc_node(nodes, node_in, wn, b, blk):
    n = nodes.shape[0]
    return pl.pallas_call(
        _node_body,
        grid=(n // blk,),
        in_specs=[
            pl.BlockSpec((blk, H), lambda i: (i, 0)),
            pl.BlockSpec((blk, 2 * H), lambda i: (i, 0)),
            pl.BlockSpec((2 * H, H), lambda i: (0, 0)),
            pl.BlockSpec((1, H), lambda i: (0, 0)),
        ],
        out_specs=pl.BlockSpec((blk, H), lambda i: (i, 0)),
        out_shape=jax.ShapeDtypeStruct((n, H), jnp.float32),
    )(nodes, node_in, wn, b.reshape(1, H))


def _node_last_body(n_ref, ni_ref, wn_ref, b_ref, wdec_ref, bdec_ref,
                    bus_ref, vtab_ref):
    n2 = n_ref[...] + jnp.maximum(
        jnp.dot(ni_ref[...], wn_ref[...],
                preferred_element_type=jnp.float32) + b_ref[...], 0.0)
    bus = jnp.dot(n2, wdec_ref[...], preferred_element_type=jnp.float32) \
        + bdec_ref[...]
    bus_ref[...] = bus
    va = bus[:, 0:1]
    vm = bus[:, 3:4]
    vre = vm * jnp.cos(va)
    vim = vm * jnp.sin(va)
    blk = vre.shape[0]
    vtab_ref[...] = jnp.concatenate(
        [vre, vim, jnp.zeros((blk, 126), jnp.float32)], axis=1)


def _tc_node_last(nodes, node_in, wn, b, w_dec, b_dec, blk):
    n = nodes.shape[0]
    return pl.pallas_call(
        _node_last_body,
        grid=(n // blk,),
        in_specs=[
            pl.BlockSpec((blk, H), lambda i: (i, 0)),
            pl.BlockSpec((blk, 2 * H), lambda i: (i, 0)),
            pl.BlockSpec((2 * H, H), lambda i: (0, 0)),
            pl.BlockSpec((1, H), lambda i: (0, 0)),
            pl.BlockSpec((H, 4), lambda i: (0, 0)),
            pl.BlockSpec((1, 4), lambda i: (0, 0)),
        ],
        out_specs=[
            pl.BlockSpec((blk, 4), lambda i: (i, 0)),
            pl.BlockSpec((blk, H), lambda i: (i, 0)),
        ],
        out_shape=[
            jax.ShapeDtypeStruct((n, 4), jnp.float32),
            jax.ShapeDtypeStruct((n, H), jnp.float32),
        ],
    )(nodes, node_in, wn, b.reshape(1, H), w_dec, b_dec.reshape(1, 4))


def _flow_body(v_ref, par_ref, out_ref):
    vi = v_ref[0]
    vj = v_ref[1]
    vi_re, vi_im = vi[:, 0:1], vi[:, 1:2]
    vj_re, vj_im = vj[:, 0:1], vj[:, 1:2]
    r = par_ref[:, 0:1]
    xx = par_ref[:, 1:2]
    b_fr = par_ref[:, 2:3]
    b_to = par_ref[:, 3:4]
    tap = par_ref[:, 4:5]
    shift = par_ref[:, 5:6]
    den = r * r + xx * xx
    y_re = r / den
    y_im = -xx / den
    cs = jnp.cos(shift)
    sn = jnp.sin(shift)
    t2 = tap * tap
    ai2 = vi_re * vi_re + vi_im * vi_im
    aj2 = vj_re * vj_re + vj_im * vj_im
    m_re = vi_re * vj_re + vi_im * vj_im
    m_im = vi_im * vj_re - vi_re * vj_im
    cy_m_re = y_re * m_re + y_im * m_im
    cy_m_im = y_re * m_im - y_im * m_re
    term_re = (cy_m_re * cs + cy_m_im * sn) / tap
    term_im = (cy_m_im * cs - cy_m_re * sn) / tap
    p_fr = y_re * ai2 / t2 - term_re
    q_fr = -(y_im + b_fr) * ai2 / t2 - term_im
    cy_n_re = y_re * m_re - y_im * m_im
    cy_n_im = -y_re * m_im - y_im * m_re
    term2_re = (cy_n_re * cs - cy_n_im * sn) / tap
    term2_im = (cy_n_im * cs + cy_n_re * sn) / tap
    p_to = y_re * aj2 - term2_re
    q_to = -(y_im + b_to) * aj2 - term2_im
    out_ref[...] = jnp.concatenate([p_to, q_to, p_fr, q_fr], axis=1)


def _tc_flow(v3, params, blk):
    e = params.shape[0]
    return pl.pallas_call(
        _flow_body,
        grid=(e // blk,),
        in_specs=[
            pl.BlockSpec((2, blk, H), lambda i: (0, i, 0)),
            pl.BlockSpec((blk, 8), lambda i: (i, 0)),
        ],
        out_specs=pl.BlockSpec((blk, 4), lambda i: (i, 0)),
        out_shape=jax.ShapeDtypeStruct((e, 4), jnp.float32),
    )(v3, params)


def _make_sc_gather(n_rows, d, e_total, cols_mode):
    ept = e_total // NTILES
    nb = ept // 128
    tail = ept - nb * 128
    stride = (n_rows // NTILES) // 8 * 8
    size = n_rows - stride * (NTILES - 1)
    assert size >= n_rows // NTILES and size % 8 == 0
    mesh = plsc.VectorSubcoreMesh(core_axis_name="c", subcore_axis_name="s")

    if cols_mode:
        out_t = jax.ShapeDtypeStruct((e_total, 2 * d), jnp.float32)
    else:
        out_t = jax.ShapeDtypeStruct((2 * e_total, d), jnp.float32)

    assert nb >= 4 and nb % 2 == 0
    scratch = [
        pltpu.VMEM_SHARED((n_rows, d), jnp.float32),
        pltpu.VMEM((128,), jnp.int32),
        pltpu.VMEM((128,), jnp.int32),
        pltpu.VMEM((128, d), jnp.float32),
        pltpu.VMEM((128, d), jnp.float32),
        pltpu.SemaphoreType.DMA,
        pltpu.SemaphoreType.DMA,
        pltpu.SemaphoreType.DMA,
        pltpu.SemaphoreType.DMA,
        pltpu.SemaphoreType.DMA,
    ]
    if tail:
        scratch += [
            pltpu.VMEM((tail,), jnp.int32),
            pltpu.VMEM((tail, d), jnp.float32),
        ]

    @functools.partial(
        pl.kernel,
        out_type=out_t,
        mesh=mesh,
        scratch_types=scratch,
    )
    def k(tab_hbm, idx_hbm, out_hbm, shared, i0, i1, r0, r1, si0, si1,
          sg, so0, so1, *tl):
        c = lax.axis_index("c")
        s = lax.axis_index("s")
        pltpu.sync_copy(
            tab_hbm.at[pl.ds(s * stride, size), :],
            shared.at[pl.ds(s * stride, size), :])
        plsc.subcore_barrier()
        base = c * e_total + s * ept
        idxb = (i0, i1)
        rows = (r0, r1)
        semi = (si0, si1)
        semo = (so0, so1)

        def out_at(b0, cnt):
            erow = b0 - c * e_total
            if cols_mode:
                return out_hbm.at[pl.ds(erow, cnt), pl.ds(c * d, d)]
            return out_hbm.at[pl.ds(b0, cnt), :]

        pltpu.async_copy(idx_hbm.at[pl.ds(base, 128)], i0, si0)
        pltpu.async_copy(idx_hbm.at[pl.ds(base + 128, 128)], i1, si1)

        def body(it, _):
            for p in range(2):
                blk = it * 2 + p
                b0 = base + blk * 128
                @pl.when(blk >= 2)
                def _():
                    pltpu.make_async_copy(
                        rows[p], out_at(b0, 128), semo[p]).wait()
                pltpu.make_async_copy(
                    idx_hbm.at[pl.ds(b0, 128)], idxb[p], semi[p]).wait()
                pltpu.async_copy(shared.at[idxb[p]], rows[p], sg).wait()
                pltpu.async_copy(rows[p], out_at(b0, 128), semo[p])

                @pl.when(blk + 2 < nb)
                def _():
                    pltpu.async_copy(
                        idx_hbm.at[pl.ds(b0 + 2 * 128, 128)],
                        idxb[p], semi[p])
            return 0

        lax.fori_loop(0, nb // 2, body, 0, unroll=False)
        for p in range(2):
            pltpu.make_async_copy(
                rows[p], out_at(base, 128), semo[p]).wait()
        if tail:
            idx_t, rows_t = tl
            b0 = base + nb * 128
            pltpu.sync_copy(idx_hbm.at[pl.ds(b0, tail)], idx_t)
            pltpu.async_copy(shared.at[idx_t], rows_t, sg).wait()
            pltpu.sync_copy(rows_t, out_at(b0, tail))

    return k


def kernel(x, edge_index_ac, edge_index_tf, branch_vals_ac, branch_vals_tf,
           W_enc, b_enc, W_e_ac, W_e_tf, msg_W, msg_b, node_W, node_b,
           W_dec, b_dec):
    n = x.shape[0]
    e_ac = branch_vals_ac.shape[0]
    e_tf = branch_vals_tf.shape[0]
    e = e_ac + e_tf
    kk = msg_W.shape[0]

    ei = jnp.concatenate([edge_index_ac, edge_index_tf], axis=1)
    idx_flat = ei.reshape(2 * e)
    dst = ei[1]
    params = jnp.stack([
        jnp.concatenate([branch_vals_ac[:, 4], branch_vals_tf[:, 2]]),
        jnp.concatenate([branch_vals_ac[:, 5], branch_vals_tf[:, 3]]),
        jnp.concatenate([branch_vals_ac[:, 2], branch_vals_tf[:, 9]]),
        jnp.concatenate([branch_vals_ac[:, 3], branch_vals_tf[:, 10]]),
        jnp.concatenate([jnp.ones((e_ac,), jnp.float32),
                         branch_vals_tf[:, 7]]),
        jnp.concatenate([jnp.zeros((e_ac,), jnp.float32),
                         branch_vals_tf[:, 8]]),
        jnp.zeros((e,), jnp.float32),
        jnp.zeros((e,), jnp.float32),
    ], axis=1)

    gather_nodes = _make_sc_gather(n, H, e, cols_mode=True)
    gather_v = _make_sc_gather(n, H, e, cols_mode=False)

    nodes = _tc_prep(x, W_enc, b_enc, 1000)
    edges = jnp.concatenate([
        _tc_enc(branch_vals_ac, W_e_ac, 512),
        _tc_enc(branch_vals_tf, W_e_tf, 512),
    ], axis=0)
    bus_out = vtab = None
    for l in range(kk):
        g256 = gather_nodes(nodes, idx_flat)
        edges = _tc_edge(edges, g256, msg_W[l, :2 * H, :],
                         msg_W[l, 2 * H:, :], msg_b[l], 512)
        agg = jax.ops.segment_sum(edges, dst, num_segments=n)
        node_in = jnp.concatenate([nodes, agg], axis=1)
        if l < kk - 1:
            nodes = _tc_node(nodes, node_in, node_W[l], node_b[l], 1000)
        else:
            bus_out, vtab = _tc_node_last(nodes, node_in, node_W[l],
                                          node_b[l], W_dec, b_dec, 1000)
    vij = gather_v(vtab, idx_flat)
    edge_preds = _tc_flow(vij.reshape(2, e, H), params, 2000)
    return bus_out, edge_preds
